# baseline (speedup 1.0000x reference)
import functools

import jax
import jax.numpy as jnp
from jax import lax
from jax.experimental import pallas as pl
from jax.experimental.pallas import tpu as pltpu
from jax.experimental.pallas import tpu_sc as plsc

N = 10000
E = 320000
D = 128
NW = 32
CPW = D // NW
SLICE = CPW * N
CH = 1600
NCHUNK = E // CH
PAIRS = CH // 32


def _sm_group(hbuf, accs, pe, w16):
    s16 = pe & 0x3FFF
    d16 = lax.shift_right_logical(pe, 14)
    vs = []
    for c in range(CPW):
        hv = plsc.load_gather(hbuf, [s16 + c * N])
        vs.append(hv * w16)
    cnt, _ = plsc.scan_count(d16)
    dup = jnp.max(cnt) - jnp.min(cnt)
    for c in range(CPW):
        cur = plsc.load_gather(accs[c], [d16])
        plsc.store_scatter(accs[c], [d16], jnp.maximum(cur, vs[c]))
    return dup, d16, vs


def _sm_repair(accs, d16, vs):
    for c in range(CPW):
        acc = accs[c]
        v = vs[c]

        def _cond(st):
            return st > 0

        def _body(st, _acc=acc, _v=v):
            cur = plsc.load_gather(_acc, [d16])
            pend = _v > cur
            plsc.store_scatter(_acc, [d16], _v, mask=pend)
            chk = plsc.load_gather(_acc, [d16])
            return jnp.max((_v > chk).astype(jnp.int32))

        lax.while_loop(_cond, _body, jnp.int32(1))


def _segmax_body(h_hbm, pe_hbm, ew_hbm, out_hbm,
                 hbuf, c0e, c1e, c2e, c3e, c0o, c1o, c2o, c3o,
                 pebuf0, pebuf1, ewbuf0, ewbuf1,
                 sem0, sem1):
    wid = lax.axis_index("s") * 2 + lax.axis_index("c")
    base = wid * SLICE
    sems = (sem0, sem1)
    pebufs = (pebuf0, pebuf1)
    ewbufs = (ewbuf0, ewbuf1)
    acc_e = (c0e, c1e, c2e, c3e)
    acc_o = (c0o, c1o, c2o, c3o)

    pltpu.sync_copy(h_hbm.at[pl.ds(base, SLICE)], hbuf)

    def _zero(i, carry):
        sl = pl.ds(i * 16, 16)
        z = jnp.zeros((16,), jnp.float32)
        for r in acc_e + acc_o:
            r[sl] = z
        return carry

    lax.fori_loop(0, N // 16, _zero, 0)

    for par in (0, 1):
        pltpu.async_copy(pe_hbm.at[pl.ds(par * CH, CH)], pebufs[par],
                         sems[par])
        pltpu.async_copy(ew_hbm.at[pl.ds(par * CH, CH)], ewbufs[par],
                         sems[par])

    def _chunk(t, carry):
        for par in (0, 1):
            tc = 2 * t + par
            off = tc * CH
            pltpu.make_async_copy(pe_hbm.at[pl.ds(off, CH)], pebufs[par],
                                  sems[par]).wait()
            pltpu.make_async_copy(ew_hbm.at[pl.ds(off, CH)], ewbufs[par],
                                  sems[par]).wait()

            NP = CH // 32

            def _ldpair(g):
                b = g * 32
                return (pebufs[par][pl.ds(b, 16)],
                        ewbufs[par][pl.ds(b, 16)],
                        pebufs[par][pl.ds(b + 16, 16)],
                        ewbufs[par][pl.ds(b + 16, 16)])

            def _pair(g, gcarry):
                pend = gcarry[0]
                d0p, d1p = gcarry[1], gcarry[2]
                vsp = gcarry[3]
                pe0, w0, pe1, w1 = gcarry[4]
                dup0, d0, vs0 = _sm_group(hbuf, acc_e, pe0, w0)
                dup1, d1, vs1 = _sm_group(hbuf, acc_o, pe1, w1)
                gn = jnp.minimum(g + 1, NP - 1)
                bn = gn * 32
                nxt = (pebufs[par][pl.ds(bn, 16)],
                       ewbufs[par][pl.ds(bn, 16)],
                       pebufs[par][pl.ds(bn + 16, 16)],
                       ewbufs[par][pl.ds(bn + 16, 16)])

                def _slow():
                    _sm_repair(acc_e, d0p, vsp[:4])
                    _sm_repair(acc_o, d1p, vsp[4:])

                lax.cond(pend > 0, _slow, lambda: None)
                return (dup0 + dup1, d0, d1, tuple(vs0) + tuple(vs1), nxt)

            zv = jnp.zeros((16,), jnp.float32)
            zi = jnp.zeros((16,), jnp.int32)
            fc = lax.fori_loop(0, NP, _pair,
                               (jnp.int32(0), zi, zi, (zv,) * 8, _ldpair(0)))

            def _slow_tail():
                _sm_repair(acc_e, fc[1], fc[3][:4])
                _sm_repair(acc_o, fc[2], fc[3][4:])

            lax.cond(fc[0] > 0, _slow_tail, lambda: None)

            @pl.when(tc + 2 < NCHUNK)
            def _():
                noff = (tc + 2) * CH
                pltpu.async_copy(pe_hbm.at[pl.ds(noff, CH)], pebufs[par],
                                 sems[par])
                pltpu.async_copy(ew_hbm.at[pl.ds(noff, CH)], ewbufs[par],
                                 sems[par])
        return carry

    lax.fori_loop(0, NCHUNK // 2, _chunk, 0)

    def _merge(i, carry):
        sl = pl.ds(i * 16, 16)
        for e, o in zip(acc_e, acc_o):
            e[sl] = jnp.maximum(e[sl], o[sl])
        return carry

    lax.fori_loop(0, N // 16, _merge, 0)
    for c in range(CPW):
        pltpu.sync_copy(acc_e[c], out_hbm.at[pl.ds(base + c * N, N)])


_segmax = functools.partial(
    pl.kernel,
    mesh=plsc.VectorSubcoreMesh(core_axis_name="c", subcore_axis_name="s"),
    out_type=jax.ShapeDtypeStruct((D * N,), jnp.float32),
    scratch_types=[
        pltpu.VMEM((SLICE,), jnp.float32),
        pltpu.VMEM((N,), jnp.float32),
        pltpu.VMEM((N,), jnp.float32),
        pltpu.VMEM((N,), jnp.float32),
        pltpu.VMEM((N,), jnp.float32),
        pltpu.VMEM((N,), jnp.float32),
        pltpu.VMEM((N,), jnp.float32),
        pltpu.VMEM((N,), jnp.float32),
        pltpu.VMEM((N,), jnp.float32),
        pltpu.VMEM((CH,), jnp.int32),
        pltpu.VMEM((CH,), jnp.int32),
        pltpu.VMEM((CH,), jnp.float32),
        pltpu.VMEM((CH,), jnp.float32),
        pltpu.SemaphoreType.DMA,
        pltpu.SemaphoreType.DMA,
    ],
    compiler_params=pltpu.CompilerParams(needs_layout_passes=False),
)(_segmax_body)


def _pack_body(z_ref, pe_ref):
    pe_ref[...] = z_ref[0:1, :] | lax.shift_left(z_ref[1:2, :], 14)


def _pack_edges(z):
    return pl.pallas_call(
        _pack_body,
        in_specs=[pl.BlockSpec((2, E), lambda: (0, 0))],
        out_specs=pl.BlockSpec((1, E), lambda: (0, 0)),
        out_shape=jax.ShapeDtypeStruct((1, E), jnp.int32),
    )(z).reshape(E)


BN = 1280


def _pre1_body(x_ref, wp_ref, bp_ref, ws_ref, b_ref, h_ref, s_ref):
    x = x_ref[...]
    h = lax.dot_general(wp_ref[...], x, (((0,), (1,)), ((), ())),
                        preferred_element_type=jnp.float32)
    h_ref[...] = jnp.maximum(h + bp_ref[...], 0.0)
    s = lax.dot_general(ws_ref[...], x, (((0,), (1,)), ((), ())),
                        preferred_element_type=jnp.float32)
    s_ref[...] = s + b_ref[...]


def _pre_body(x_ref, wp_ref, bp_ref, ws_ref, b_ref, h_ref, s_ref):
    x = x_ref[...]
    h = lax.dot_general(wp_ref[...], x, (((0,), (0,)), ((), ())),
                        preferred_element_type=jnp.float32)
    h_ref[...] = jnp.maximum(h + bp_ref[...], 0.0)
    s = lax.dot_general(ws_ref[...], x, (((0,), (0,)), ((), ())),
                        preferred_element_type=jnp.float32)
    s_ref[...] = s + b_ref[...]


def _tc_pre(x, wp, bp_col, ws, b_col, first):
    body = _pre1_body if first else _pre_body
    in_spec = (pl.BlockSpec((BN, D), lambda i: (i, 0)) if first
               else pl.BlockSpec((D, BN), lambda i: (0, i)))
    return pl.pallas_call(
        body,
        grid=(-(-N // BN),),
        in_specs=[
            in_spec,
            pl.BlockSpec((D, D), lambda i: (0, 0)),
            pl.BlockSpec((D, 1), lambda i: (0, 0)),
            pl.BlockSpec((D, D), lambda i: (0, 0)),
            pl.BlockSpec((D, 1), lambda i: (0, 0)),
        ],
        out_specs=[
            pl.BlockSpec((D, BN), lambda i: (0, i)),
            pl.BlockSpec((D, BN), lambda i: (0, i)),
        ],
        out_shape=[
            jax.ShapeDtypeStruct((D, N), jnp.float32),
            jax.ShapeDtypeStruct((D, N), jnp.float32),
        ],
    )(x, wp, bp_col, ws, b_col)


def _mid_body(s_ref, n_ref, wn_ref, wp_ref, bp_ref, ws_ref, b_ref,
              h_ref, s2_ref):
    m = lax.dot_general(wn_ref[...], n_ref[...], (((0,), (0,)), ((), ())),
                        preferred_element_type=jnp.float32)
    x = jnp.maximum(s_ref[...] + m, 0.0)
    h = lax.dot_general(wp_ref[...], x, (((0,), (0,)), ((), ())),
                        preferred_element_type=jnp.float32)
    h_ref[...] = jnp.maximum(h + bp_ref[...], 0.0)
    s2 = lax.dot_general(ws_ref[...], x, (((0,), (0,)), ((), ())),
                         preferred_element_type=jnp.float32)
    s2_ref[...] = s2 + b_ref[...]


def _tc_mid(s_t, neigh_t, wn, wp, bp_col, ws, b_col):
    return pl.pallas_call(
        _mid_body,
        grid=(-(-N // BN),),
        in_specs=[
            pl.BlockSpec((D, BN), lambda i: (0, i)),
            pl.BlockSpec((D, BN), lambda i: (0, i)),
            pl.BlockSpec((D, D), lambda i: (0, 0)),
            pl.BlockSpec((D, D), lambda i: (0, 0)),
            pl.BlockSpec((D, 1), lambda i: (0, 0)),
            pl.BlockSpec((D, D), lambda i: (0, 0)),
            pl.BlockSpec((D, 1), lambda i: (0, 0)),
        ],
        out_specs=[
            pl.BlockSpec((D, BN), lambda i: (0, i)),
            pl.BlockSpec((D, BN), lambda i: (0, i)),
        ],
        out_shape=[
            jax.ShapeDtypeStruct((D, N), jnp.float32),
            jax.ShapeDtypeStruct((D, N), jnp.float32),
        ],
    )(s_t, neigh_t, wn, wp, bp_col, ws, b_col)


def _post_body(s_ref, n_ref, wn_ref, o_ref):
    m = lax.dot_general(wn_ref[...], n_ref[...], (((0,), (0,)), ((), ())),
                        preferred_element_type=jnp.float32)
    o_ref[...] = jnp.maximum(s_ref[...] + m, 0.0)


def _tc_post(s_t, neigh_t, wn):
    return pl.pallas_call(
        _post_body,
        grid=(-(-N // BN),),
        in_specs=[
            pl.BlockSpec((D, BN), lambda i: (0, i)),
            pl.BlockSpec((D, BN), lambda i: (0, i)),
            pl.BlockSpec((D, D), lambda i: (0, 0)),
        ],
        out_specs=pl.BlockSpec((D, BN), lambda i: (0, i)),
        out_shape=jax.ShapeDtypeStruct((D, N), jnp.float32),
    )(s_t, neigh_t, wn)


BF = 1280


def _final_body(xi_ref, xj_ref, adj_ref, hd_ref):
    a = xi_ref[...]
    bb = xj_ref[...]
    adj_ref[...] = lax.dot_general(a, bb, (((0,), (0,)), ((), ())),
                                   preferred_element_type=jnp.float32)

    @pl.when(pl.program_id(1) == 0)
    def _():
        hd_ref[...] = a.T


def _tc_final(x_t):
    return pl.pallas_call(
        _final_body,
        grid=(-(-N // BF), -(-N // BF)),
        in_specs=[
            pl.BlockSpec((D, BF), lambda i, j: (0, i)),
            pl.BlockSpec((D, BF), lambda i, j: (0, j)),
        ],
        out_specs=[
            pl.BlockSpec((BF, BF), lambda i, j: (i, j)),
            pl.BlockSpec((BF, D), lambda i, j: (i, 0)),
        ],
        out_shape=[
            jax.ShapeDtypeStruct((N, N), jnp.float32),
            jax.ShapeDtypeStruct((N, D), jnp.float32),
        ],
    )(x_t, x_t)


def kernel(z, feat, edge_weight,
           Wp1, bp1, Ws1, Wn1, b1,
           Wp2, bp2, Ws2, Wn2, b2,
           Wp3, bp3, Ws3, Wn3, b3):
    pe = _pack_edges(z)
    h_t, s_t = _tc_pre(feat, Wp1, bp1.reshape(D, 1), Ws1, b1.reshape(D, 1),
                       first=True)
    n_t = _segmax(h_t.reshape(-1), pe, edge_weight).reshape(D, N)
    for wn, wp, bp, ws, b in ((Wn1, Wp2, bp2, Ws2, b2),
                              (Wn2, Wp3, bp3, Ws3, b3)):
        h_t, s_t = _tc_mid(s_t, n_t, wn, wp, bp.reshape(D, 1), ws,
                           b.reshape(D, 1))
        n_t = _segmax(h_t.reshape(-1), pe, edge_weight).reshape(D, N)
    x = _tc_post(s_t, n_t, Wn3)
    adj, hd = _tc_final(x)
    return (hd, adj)


# speedup vs baseline: 1.0127x; 1.0127x over previous
import functools

import jax
import jax.numpy as jnp
from jax import lax
from jax.experimental import pallas as pl
from jax.experimental.pallas import tpu as pltpu
from jax.experimental.pallas import tpu_sc as plsc

N = 10000
E = 320000
D = 128
NW = 32
CPW = D // NW
SLICE = CPW * N
CH = 1600
NCHUNK = E // CH


def _sm_group(hbuf, accs, pe, w16):
    s16 = pe & 0x3FFF
    d16 = lax.shift_right_logical(pe, 14)
    vs = []
    for c in range(CPW):
        hv = plsc.load_gather(hbuf, [s16 + c * N])
        vs.append(hv * w16)
    cnt, _ = plsc.scan_count(d16)
    dup = jnp.max(cnt) - jnp.min(cnt)
    for c in range(CPW):
        cur = plsc.load_gather(accs[c], [d16])
        plsc.store_scatter(accs[c], [d16], jnp.maximum(cur, vs[c]))
    return dup, d16, vs


def _sm_repair(accs, d16, vs):
    for c in range(CPW):
        acc = accs[c]
        v = vs[c]

        def _cond(st):
            return st > 0

        def _body(st, _acc=acc, _v=v):
            cur = plsc.load_gather(_acc, [d16])
            pend = _v > cur
            plsc.store_scatter(_acc, [d16], _v, mask=pend)
            chk = plsc.load_gather(_acc, [d16])
            return jnp.max((_v > chk).astype(jnp.int32))

        lax.while_loop(_cond, _body, jnp.int32(1))


def _segmax_body(h_hbm, pe_hbm, ew_hbm, out_hbm,
                 hbuf, c0e, c1e, c2e, c3e, c0o, c1o, c2o, c3o,
                 pebuf0, pebuf1, ewbuf0, ewbuf1,
                 sem0, sem1):
    wid = lax.axis_index("s") * 2 + lax.axis_index("c")
    base = wid * SLICE
    sems = (sem0, sem1)
    pebufs = (pebuf0, pebuf1)
    ewbufs = (ewbuf0, ewbuf1)
    acc_e = (c0e, c1e, c2e, c3e)
    acc_o = (c0o, c1o, c2o, c3o)

    pltpu.sync_copy(h_hbm.at[pl.ds(base, SLICE)], hbuf)

    def _zero(i, carry):
        sl = pl.ds(i * 16, 16)
        z = jnp.zeros((16,), jnp.float32)
        for r in acc_e + acc_o:
            r[sl] = z
        return carry

    lax.fori_loop(0, N // 16, _zero, 0)

    for par in (0, 1):
        pltpu.async_copy(pe_hbm.at[pl.ds(par * CH, CH)], pebufs[par],
                         sems[par])
        pltpu.async_copy(ew_hbm.at[pl.ds(par * CH, CH)], ewbufs[par],
                         sems[par])

    def _chunk(t, carry):
        for par in (0, 1):
            tc = 2 * t + par
            off = tc * CH
            pltpu.make_async_copy(pe_hbm.at[pl.ds(off, CH)], pebufs[par],
                                  sems[par]).wait()
            pltpu.make_async_copy(ew_hbm.at[pl.ds(off, CH)], ewbufs[par],
                                  sems[par]).wait()

            def _pair(g, gcarry):
                pend = gcarry[0]
                d0p, d1p = gcarry[1], gcarry[2]
                vsp = gcarry[3]
                b = g * 32
                pe0 = pebufs[par][pl.ds(b, 16)]
                w0 = ewbufs[par][pl.ds(b, 16)]
                pe1 = pebufs[par][pl.ds(b + 16, 16)]
                w1 = ewbufs[par][pl.ds(b + 16, 16)]
                dup0, d0, vs0 = _sm_group(hbuf, acc_e, pe0, w0)
                dup1, d1, vs1 = _sm_group(hbuf, acc_o, pe1, w1)

                def _slow():
                    _sm_repair(acc_e, d0p, vsp[:4])
                    _sm_repair(acc_o, d1p, vsp[4:])

                lax.cond(pend > 0, _slow, lambda: None)
                return (dup0 + dup1, d0, d1, tuple(vs0) + tuple(vs1))

            zv = jnp.zeros((16,), jnp.float32)
            zi = jnp.zeros((16,), jnp.int32)
            fc = lax.fori_loop(0, CH // 32, _pair,
                               (jnp.int32(0), zi, zi, (zv,) * 8))

            def _slow_tail():
                _sm_repair(acc_e, fc[1], fc[3][:4])
                _sm_repair(acc_o, fc[2], fc[3][4:])

            lax.cond(fc[0] > 0, _slow_tail, lambda: None)

            @pl.when(tc + 2 < NCHUNK)
            def _():
                noff = (tc + 2) * CH
                pltpu.async_copy(pe_hbm.at[pl.ds(noff, CH)], pebufs[par],
                                 sems[par])
                pltpu.async_copy(ew_hbm.at[pl.ds(noff, CH)], ewbufs[par],
                                 sems[par])
        return carry

    lax.fori_loop(0, NCHUNK // 2, _chunk, 0)

    def _merge(i, carry):
        sl = pl.ds(i * 16, 16)
        for e, o in zip(acc_e, acc_o):
            e[sl] = jnp.maximum(e[sl], o[sl])
        return carry

    lax.fori_loop(0, N // 16, _merge, 0)
    for c in range(CPW):
        pltpu.sync_copy(acc_e[c], out_hbm.at[pl.ds(base + c * N, N)])


_segmax = functools.partial(
    pl.kernel,
    mesh=plsc.VectorSubcoreMesh(core_axis_name="c", subcore_axis_name="s"),
    out_type=jax.ShapeDtypeStruct((D * N,), jnp.float32),
    scratch_types=[
        pltpu.VMEM((SLICE,), jnp.float32),
        pltpu.VMEM((N,), jnp.float32),
        pltpu.VMEM((N,), jnp.float32),
        pltpu.VMEM((N,), jnp.float32),
        pltpu.VMEM((N,), jnp.float32),
        pltpu.VMEM((N,), jnp.float32),
        pltpu.VMEM((N,), jnp.float32),
        pltpu.VMEM((N,), jnp.float32),
        pltpu.VMEM((N,), jnp.float32),
        pltpu.VMEM((CH,), jnp.int32),
        pltpu.VMEM((CH,), jnp.int32),
        pltpu.VMEM((CH,), jnp.float32),
        pltpu.VMEM((CH,), jnp.float32),
        pltpu.SemaphoreType.DMA,
        pltpu.SemaphoreType.DMA,
    ],
    compiler_params=pltpu.CompilerParams(needs_layout_passes=False),
)(_segmax_body)


def _pack_body(z_ref, pe_ref):
    pe_ref[...] = z_ref[0:1, :] | lax.shift_left(z_ref[1:2, :], 14)


def _pack_edges(z):
    return pl.pallas_call(
        _pack_body,
        in_specs=[pl.BlockSpec((2, E), lambda: (0, 0))],
        out_specs=pl.BlockSpec((1, E), lambda: (0, 0)),
        out_shape=jax.ShapeDtypeStruct((1, E), jnp.int32),
    )(z).reshape(E)


BN = 1280


def _pre1_body(x_ref, wp_ref, bp_ref, ws_ref, b_ref, h_ref, s_ref):
    x = x_ref[...]
    h = lax.dot_general(wp_ref[...], x, (((0,), (1,)), ((), ())),
                        preferred_element_type=jnp.float32)
    h_ref[...] = jnp.maximum(h + bp_ref[...], 0.0)
    s = lax.dot_general(ws_ref[...], x, (((0,), (1,)), ((), ())),
                        preferred_element_type=jnp.float32)
    s_ref[...] = s + b_ref[...]


def _pre_body(x_ref, wp_ref, bp_ref, ws_ref, b_ref, h_ref, s_ref):
    x = x_ref[...]
    h = lax.dot_general(wp_ref[...], x, (((0,), (0,)), ((), ())),
                        preferred_element_type=jnp.float32)
    h_ref[...] = jnp.maximum(h + bp_ref[...], 0.0)
    s = lax.dot_general(ws_ref[...], x, (((0,), (0,)), ((), ())),
                        preferred_element_type=jnp.float32)
    s_ref[...] = s + b_ref[...]


def _tc_pre(x, wp, bp_col, ws, b_col, first):
    body = _pre1_body if first else _pre_body
    in_spec = (pl.BlockSpec((BN, D), lambda i: (i, 0)) if first
               else pl.BlockSpec((D, BN), lambda i: (0, i)))
    return pl.pallas_call(
        body,
        grid=(-(-N // BN),),
        in_specs=[
            in_spec,
            pl.BlockSpec((D, D), lambda i: (0, 0)),
            pl.BlockSpec((D, 1), lambda i: (0, 0)),
            pl.BlockSpec((D, D), lambda i: (0, 0)),
            pl.BlockSpec((D, 1), lambda i: (0, 0)),
        ],
        out_specs=[
            pl.BlockSpec((D, BN), lambda i: (0, i)),
            pl.BlockSpec((D, BN), lambda i: (0, i)),
        ],
        out_shape=[
            jax.ShapeDtypeStruct((D, N), jnp.float32),
            jax.ShapeDtypeStruct((D, N), jnp.float32),
        ],
    )(x, wp, bp_col, ws, b_col)


def _mid_body(s_ref, n_ref, wn_ref, wp_ref, bp_ref, ws_ref, b_ref,
              h_ref, s2_ref):
    m = lax.dot_general(wn_ref[...], n_ref[...], (((0,), (0,)), ((), ())),
                        preferred_element_type=jnp.float32)
    x = jnp.maximum(s_ref[...] + m, 0.0)
    h = lax.dot_general(wp_ref[...], x, (((0,), (0,)), ((), ())),
                        preferred_element_type=jnp.float32)
    h_ref[...] = jnp.maximum(h + bp_ref[...], 0.0)
    s2 = lax.dot_general(ws_ref[...], x, (((0,), (0,)), ((), ())),
                         preferred_element_type=jnp.float32)
    s2_ref[...] = s2 + b_ref[...]


def _tc_mid(s_t, neigh_t, wn, wp, bp_col, ws, b_col):
    return pl.pallas_call(
        _mid_body,
        grid=(-(-N // BN),),
        in_specs=[
            pl.BlockSpec((D, BN), lambda i: (0, i)),
            pl.BlockSpec((D, BN), lambda i: (0, i)),
            pl.BlockSpec((D, D), lambda i: (0, 0)),
            pl.BlockSpec((D, D), lambda i: (0, 0)),
            pl.BlockSpec((D, 1), lambda i: (0, 0)),
            pl.BlockSpec((D, D), lambda i: (0, 0)),
            pl.BlockSpec((D, 1), lambda i: (0, 0)),
        ],
        out_specs=[
            pl.BlockSpec((D, BN), lambda i: (0, i)),
            pl.BlockSpec((D, BN), lambda i: (0, i)),
        ],
        out_shape=[
            jax.ShapeDtypeStruct((D, N), jnp.float32),
            jax.ShapeDtypeStruct((D, N), jnp.float32),
        ],
    )(s_t, neigh_t, wn, wp, bp_col, ws, b_col)


def _post_body(s_ref, n_ref, wn_ref, o_ref):
    m = lax.dot_general(wn_ref[...], n_ref[...], (((0,), (0,)), ((), ())),
                        preferred_element_type=jnp.float32)
    o_ref[...] = jnp.maximum(s_ref[...] + m, 0.0)


def _tc_post(s_t, neigh_t, wn):
    return pl.pallas_call(
        _post_body,
        grid=(-(-N // BN),),
        in_specs=[
            pl.BlockSpec((D, BN), lambda i: (0, i)),
            pl.BlockSpec((D, BN), lambda i: (0, i)),
            pl.BlockSpec((D, D), lambda i: (0, 0)),
        ],
        out_specs=pl.BlockSpec((D, BN), lambda i: (0, i)),
        out_shape=jax.ShapeDtypeStruct((D, N), jnp.float32),
    )(s_t, neigh_t, wn)


BF = 1280


def _final_body(xi_ref, xj_ref, adj_ref, hd_ref):
    a = xi_ref[...]
    bb = xj_ref[...]
    adj_ref[...] = lax.dot_general(a, bb, (((0,), (0,)), ((), ())),
                                   preferred_element_type=jnp.float32)

    @pl.when(pl.program_id(1) == 0)
    def _():
        hd_ref[...] = a.T


def _tc_final(x_t):
    return pl.pallas_call(
        _final_body,
        grid=(-(-N // BF), -(-N // BF)),
        in_specs=[
            pl.BlockSpec((D, BF), lambda i, j: (0, i)),
            pl.BlockSpec((D, BF), lambda i, j: (0, j)),
        ],
        out_specs=[
            pl.BlockSpec((BF, BF), lambda i, j: (i, j)),
            pl.BlockSpec((BF, D), lambda i, j: (i, 0)),
        ],
        out_shape=[
            jax.ShapeDtypeStruct((N, N), jnp.float32),
            jax.ShapeDtypeStruct((N, D), jnp.float32),
        ],
    )(x_t, x_t)


def kernel(z, feat, edge_weight,
           Wp1, bp1, Ws1, Wn1, b1,
           Wp2, bp2, Ws2, Wn2, b2,
           Wp3, bp3, Ws3, Wn3, b3):
    pe = _pack_edges(z)
    h_t, s_t = _tc_pre(feat, Wp1, bp1.reshape(D, 1), Ws1, b1.reshape(D, 1),
                       first=True)
    n_t = _segmax(h_t.reshape(-1), pe, edge_weight).reshape(D, N)
    for wn, wp, bp, ws, b in ((Wn1, Wp2, bp2, Ws2, b2),
                              (Wn2, Wp3, bp3, Ws3, b3)):
        h_t, s_t = _tc_mid(s_t, n_t, wn, wp, bp.reshape(D, 1), ws,
                           b.reshape(D, 1))
        n_t = _segmax(h_t.reshape(-1), pe, edge_weight).reshape(D, N)
    x = _tc_post(s_t, n_t, Wn3)
    adj, hd = _tc_final(x)
    return (hd, adj)
